# Initial kernel scaffold; baseline (speedup 1.0000x reference)
import jax
import jax.numpy as jnp
from jax.experimental import pallas as pl


def kernel(x, W, b):
    raise NotImplementedError("write your pallas kernel here")


# speedup vs baseline: 1.0090x; 1.0090x over previous
import jax
import jax.numpy as jnp
from jax.experimental import pallas as pl

IN_F = 1024
BN = 2048


def _linear_block(x_ref, w_ref, b_ref, o_ref):
    xb = x_ref[...].astype(jnp.bfloat16)
    wb = w_ref[...].astype(jnp.bfloat16)
    acc = jax.lax.dot_general(
        xb, wb,
        dimension_numbers=(((1,), (1,)), ((), ())),
        preferred_element_type=jnp.float32,
    )
    o_ref[...] = acc + b_ref[...]


def kernel(x, W, b):
    batch, in_f = x.shape
    out_f = W.shape[0]
    grid = (out_f + BN - 1) // BN
    b2 = b.reshape(1, out_f)
    return pl.pallas_call(
        _linear_block,
        grid=(grid,),
        in_specs=[
            pl.BlockSpec((batch, in_f), lambda j: (0, 0)),
            pl.BlockSpec((BN, in_f), lambda j: (j, 0)),
            pl.BlockSpec((1, BN), lambda j: (0, j)),
        ],
        out_specs=pl.BlockSpec((batch, BN), lambda j: (0, j)),
        out_shape=jax.ShapeDtypeStruct((batch, out_f), jnp.float32),
    )(x, W, b2)
